# baseline (speedup 1.0000x reference)
import functools

import jax
import jax.numpy as jnp
from jax import lax
from jax.experimental import pallas as pl
from jax.experimental.pallas import tpu as pltpu
from jax.experimental.pallas import tpu_sc as plsc

N = 10000
DI = 128
DO = 128

NC = 2
NS = 16
NW = NC * NS
CHUNK = 128
NSEC = 2

N_PAD = 10112
ROWS_PER_SUB = N_PAD // NS


def _sc_aggregate(x, src4, dst4):
    sec = src4.shape[2]
    mesh = plsc.VectorSubcoreMesh(core_axis_name="c", subcore_axis_name="s")

    assert sec % 2 == 0
    @functools.partial(
        pl.kernel,
        out_type=jax.ShapeDtypeStruct((NC, N_PAD, DI), jnp.float32),
        mesh=mesh,
        scratch_types=[
            pltpu.VMEM((2, sec, CHUNK), jnp.int32),
            pltpu.VMEM((2 * CHUNK, DI), jnp.float32),
            pltpu.VMEM_SHARED((N_PAD, DI), jnp.float32),
            pltpu.SemaphoreType.DMA,
            pltpu.SemaphoreType.DMA,
        ],
    )
    def k(x_hbm, src_hbm, dst_hbm, out_hbm,
          idx_v, rows_v, agg_sh, sem_a, sem_b):
        cid = lax.axis_index("c")
        sid = lax.axis_index("s")
        wid = cid * NS + sid
        my_rows = pl.ds(sid * ROWS_PER_SUB, ROWS_PER_SUB)
        zv = jnp.zeros((16,), jnp.float32)

        def zbody(r, carry):
            for l in range(DI // 16):
                rows_v[r, pl.ds(l * 16, 16)] = zv
            return carry

        lax.fori_loop(0, 2 * CHUNK, zbody, 0, unroll=False)
        base = sid * ROWS_PER_SUB
        for off in range(0, ROWS_PER_SUB, 2 * CHUNK):
            nrows = min(2 * CHUNK, ROWS_PER_SUB - off)
            pltpu.sync_copy(rows_v.at[pl.ds(0, nrows)],
                            agg_sh.at[pl.ds(base + off, nrows)])
        plsc.subcore_barrier()

        bufs = ((pl.ds(0, CHUNK), sem_a), (pl.ds(CHUNK, CHUNK), sem_b))
        for h in range(NSEC):
            pltpu.sync_copy(src_hbm.at[wid, h], idx_v.at[0])
            pltpu.sync_copy(dst_hbm.at[wid, h], idx_v.at[1])
            pltpu.async_copy(x_hbm.at[idx_v.at[0, 0]], rows_v.at[bufs[0][0]],
                             sem_a)

            def body(i, carry):
                for p in range(2):
                    j = 2 * i + p
                    half, sem = bufs[p]
                    nhalf, nsem = bufs[1 - p]

                    @pl.when(j + 1 < sec)
                    def _():
                        pltpu.async_copy(x_hbm.at[idx_v.at[0, j + 1]],
                                         rows_v.at[nhalf], nsem)

                    pltpu.make_async_copy(x_hbm.at[idx_v.at[0, j]],
                                          rows_v.at[half], sem).wait()
                    pltpu.sync_copy(rows_v.at[half], agg_sh.at[idx_v.at[1, j]],
                                    add=True)
                return carry

            lax.fori_loop(0, sec // 2, body, 0, unroll=False)

        plsc.subcore_barrier()
        pltpu.sync_copy(agg_sh.at[my_rows], out_hbm.at[cid].at[my_rows])

    return k(x, src4, dst4)


def _tc_mlp(x, parts, eps, W1, b1, g1, be1, W2, b2, g2, be2):
    def body(x_ref, p_ref, eps_ref, W1_ref, b1_ref, g1_ref, be1_ref,
             W2_ref, b2_ref, g2_ref, be2_ref, o_ref):
        agg = p_ref[0, :N, :] + p_ref[1, :N, :]
        h = (1.0 + eps_ref[0]) * x_ref[...] + agg
        y = jnp.dot(h, W1_ref[...], preferred_element_type=jnp.float32,
                    precision=lax.Precision.DEFAULT) + b1_ref[...]
        mu = jnp.mean(y, axis=0, keepdims=True)
        yc = y - mu
        var = jnp.mean(yc * yc, axis=0, keepdims=True)
        y = g1_ref[...] * yc * lax.rsqrt(var + 1e-5) + be1_ref[...]
        y = jnp.maximum(y, 0.0)
        z = jnp.dot(y, W2_ref[...], preferred_element_type=jnp.float32,
                    precision=lax.Precision.DEFAULT) + b2_ref[...]
        mu2 = jnp.mean(z, axis=0, keepdims=True)
        zc = z - mu2
        var2 = jnp.mean(zc * zc, axis=0, keepdims=True)
        z = g2_ref[...] * zc * lax.rsqrt(var2 + 1e-5) + be2_ref[...]
        o_ref[...] = jnp.maximum(z, 0.0)

    return pl.pallas_call(
        body,
        out_shape=jax.ShapeDtypeStruct((N, DO), jnp.float32),
    )(x, parts, eps, W1, b1, g1, be1, W2, b2, g2, be2)


def kernel(x, edge_index, eps, W1, b1, gamma1, beta1, W2, b2, gamma2, beta2):
    dst = edge_index[0].astype(jnp.int32)
    src = edge_index[1].astype(jnp.int32)
    e = dst.shape[0]
    epw = -(-e // NW)
    cpw = -(-epw // CHUNK)
    cpw += (-cpw) % (2 * NSEC)
    sec = cpw // NSEC
    e_pad = NW * cpw * CHUNK
    pad = e_pad - e
    pad_dst = N + jnp.arange(pad, dtype=jnp.int32) % (N_PAD - N)
    pad_src = jnp.arange(pad, dtype=jnp.int32) % N
    src4 = jnp.concatenate([src, pad_src]).reshape(NW, NSEC, sec, CHUNK)
    dst4 = jnp.concatenate([dst, pad_dst]).reshape(NW, NSEC, sec, CHUNK)
    parts = _sc_aggregate(x, src4, dst4)
    return _tc_mlp(x, parts, eps, W1, b1, gamma1, beta1, W2, b2, gamma2, beta2)


# speedup vs baseline: 1.0774x; 1.0774x over previous
import functools

import jax
import jax.numpy as jnp
from jax import lax
from jax.experimental import pallas as pl
from jax.experimental.pallas import tpu as pltpu
from jax.experimental.pallas import tpu_sc as plsc

N = 10000
DI = 128
DO = 128

NC = 2
NS = 16
NW = NC * NS
CHUNK = 128
NSEC = 2

N_PAD = 10112
ROWS_PER_SUB = N_PAD // NS


SEC = 40


def _sc_aggregate(x, ec):
    ct = ec.shape[1]
    blocks, tail = ct // 8, ct % 8
    bb, extra = blocks // NW, blocks % NW
    nbig, blen, slen = extra, 8 * (bb + 1), 8 * bb
    assert blen <= 2 * SEC and slen in (blen - 8, blen) and tail % 2 == 0
    assert tail == 0 or nbig < NW
    mesh = plsc.VectorSubcoreMesh(core_axis_name="c", subcore_axis_name="s")

    @functools.partial(
        pl.kernel,
        out_type=jax.ShapeDtypeStruct((NC, N_PAD, DI), jnp.float32),
        mesh=mesh,
        scratch_types=[
            pltpu.VMEM((2, SEC, CHUNK), jnp.int32),
            pltpu.VMEM((2 * CHUNK, DI), jnp.float32),
            pltpu.VMEM_SHARED((N_PAD, DI), jnp.float32),
            pltpu.SemaphoreType.DMA,
            pltpu.SemaphoreType.DMA,
        ],
    )
    def k(x_hbm, ec_hbm, out_hbm, idx_v, rows_v, agg_sh, sem_a, sem_b):
        cid = lax.axis_index("c")
        sid = lax.axis_index("s")
        wid = cid * NS + sid
        my_rows = pl.ds(sid * ROWS_PER_SUB, ROWS_PER_SUB)
        zv = jnp.zeros((16,), jnp.float32)

        def zbody(r, carry):
            for l in range(DI // 16):
                rows_v[r, pl.ds(l * 16, 16)] = zv
            return carry

        lax.fori_loop(0, 2 * CHUNK, zbody, 0, unroll=False)
        base = sid * ROWS_PER_SUB
        for off in range(0, ROWS_PER_SUB, 2 * CHUNK):
            nrows = min(2 * CHUNK, ROWS_PER_SUB - off)
            pltpu.sync_copy(rows_v.at[pl.ds(0, nrows)],
                            agg_sh.at[pl.ds(base + off, nrows)])
        plsc.subcore_barrier()

        bufs = ((pl.ds(0, CHUNK), sem_a), (pl.ds(CHUNK, CHUNK), sem_b))

        def section(a, n):
            a = pl.multiple_of(a, 8)
            pltpu.sync_copy(ec_hbm.at[1, pl.ds(a, n)], idx_v.at[0, pl.ds(0, n)])
            pltpu.sync_copy(ec_hbm.at[0, pl.ds(a, n)], idx_v.at[1, pl.ds(0, n)])
            pltpu.async_copy(x_hbm.at[idx_v.at[0, 0]], rows_v.at[bufs[0][0]],
                             sem_a)

            def body(i, carry):
                for p in range(2):
                    j = 2 * i + p
                    half, sem = bufs[p]
                    nhalf, nsem = bufs[1 - p]

                    @pl.when(j + 1 < n)
                    def _():
                        pltpu.async_copy(x_hbm.at[idx_v.at[0, j + 1]],
                                         rows_v.at[nhalf], nsem)

                    pltpu.make_async_copy(x_hbm.at[idx_v.at[0, j]],
                                          rows_v.at[half], sem).wait()
                    pltpu.sync_copy(rows_v.at[half], agg_sh.at[idx_v.at[1, j]],
                                    add=True)
                return carry

            lax.fori_loop(0, n // 2, body, 0, unroll=False)

        @pl.when(wid < nbig)
        def _():
            start = blen * wid
            for h in range(0, blen, SEC):
                section(start + h, min(SEC, blen - h))

        @pl.when(wid >= nbig)
        def _():
            start = slen * wid + (blen - slen) * nbig
            for h in range(0, slen, SEC):
                section(start + h, min(SEC, slen - h))

        if tail:
            @pl.when(wid == nbig)
            def _():
                section(ct - tail, tail)

        plsc.subcore_barrier()
        pltpu.sync_copy(agg_sh.at[my_rows], out_hbm.at[cid].at[my_rows])

    return k(x, ec)


def _tc_mlp(x, parts, eps, W1, b1, g1, be1, W2, b2, g2, be2):
    def body(x_ref, p_ref, eps_ref, W1_ref, b1_ref, g1_ref, be1_ref,
             W2_ref, b2_ref, g2_ref, be2_ref, o_ref):
        agg = p_ref[0, :N, :] + p_ref[1, :N, :]
        h = (1.0 + eps_ref[0]) * x_ref[...] + agg
        y = jnp.dot(h, W1_ref[...], preferred_element_type=jnp.float32,
                    precision=lax.Precision.DEFAULT) + b1_ref[...]
        mu = jnp.mean(y, axis=0, keepdims=True)
        yc = y - mu
        var = jnp.mean(yc * yc, axis=0, keepdims=True)
        y = g1_ref[...] * yc * lax.rsqrt(var + 1e-5) + be1_ref[...]
        y = jnp.maximum(y, 0.0)
        z = jnp.dot(y, W2_ref[...], preferred_element_type=jnp.float32,
                    precision=lax.Precision.DEFAULT) + b2_ref[...]
        mu2 = jnp.mean(z, axis=0, keepdims=True)
        zc = z - mu2
        var2 = jnp.mean(zc * zc, axis=0, keepdims=True)
        z = g2_ref[...] * zc * lax.rsqrt(var2 + 1e-5) + be2_ref[...]
        o_ref[...] = jnp.maximum(z, 0.0)

    return pl.pallas_call(
        body,
        out_shape=jax.ShapeDtypeStruct((N, DO), jnp.float32),
    )(x, parts, eps, W1, b1, g1, be1, W2, b2, g2, be2)


def kernel(x, edge_index, eps, W1, b1, gamma1, beta1, W2, b2, gamma2, beta2):
    e = edge_index.shape[1]
    assert e % CHUNK == 0
    ec = edge_index.astype(jnp.int32).reshape(2, e // CHUNK, CHUNK)
    parts = _sc_aggregate(x, ec)
    return _tc_mlp(x, parts, eps, W1, b1, gamma1, beta1, W2, b2, gamma2, beta2)


# speedup vs baseline: 1.0869x; 1.0089x over previous
import functools

import jax
import jax.numpy as jnp
from jax import lax
from jax.experimental import pallas as pl
from jax.experimental.pallas import tpu as pltpu
from jax.experimental.pallas import tpu_sc as plsc

N = 10000
DI = 128
DO = 128

NC = 2
NS = 16
NW = NC * NS
CHUNK = 128
NSEC = 2

N_PAD = 10112
ROWS_PER_SUB = N_PAD // NS


SEC = 40


def _sc_aggregate(x, ec):
    ct = ec.shape[1]
    blocks, tail = ct // 8, ct % 8
    bb, extra = blocks // NW, blocks % NW
    nbig, blen, slen = extra, 8 * (bb + 1), 8 * bb
    assert blen <= 2 * SEC and slen in (blen - 8, blen) and tail % 2 == 0
    assert tail == 0 or nbig < NW
    mesh = plsc.VectorSubcoreMesh(core_axis_name="c", subcore_axis_name="s")

    @functools.partial(
        pl.kernel,
        out_type=jax.ShapeDtypeStruct((NC, N_PAD, DI), jnp.float32),
        mesh=mesh,
        scratch_types=[
            pltpu.VMEM((2, SEC, CHUNK), jnp.int32),
            pltpu.VMEM((2 * CHUNK, DI), jnp.float32),
            pltpu.VMEM_SHARED((N_PAD, DI), jnp.float32),
            pltpu.SemaphoreType.DMA,
            pltpu.SemaphoreType.DMA,
        ],
    )
    def k(x_hbm, ec_hbm, out_hbm, idx_v, rows_v, agg_sh, sem_a, sem_b):
        cid = lax.axis_index("c")
        sid = lax.axis_index("s")
        wid = cid * NS + sid
        my_rows = pl.ds(sid * ROWS_PER_SUB, ROWS_PER_SUB)
        zv = jnp.zeros((16,), jnp.float32)

        def zbody(r, carry):
            for l in range(DI // 16):
                rows_v[r, pl.ds(l * 16, 16)] = zv
            return carry

        lax.fori_loop(0, 2 * CHUNK, zbody, 0, unroll=False)
        base = sid * ROWS_PER_SUB
        for off in range(0, ROWS_PER_SUB, 2 * CHUNK):
            nrows = min(2 * CHUNK, ROWS_PER_SUB - off)
            pltpu.sync_copy(rows_v.at[pl.ds(0, nrows)],
                            agg_sh.at[pl.ds(base + off, nrows)])
        plsc.subcore_barrier()

        bufs = ((pl.ds(0, CHUNK), sem_a), (pl.ds(CHUNK, CHUNK), sem_b))

        def section(a, n):
            a = pl.multiple_of(a, 8)
            pltpu.sync_copy(ec_hbm.at[1, pl.ds(a, n)], idx_v.at[0, pl.ds(0, n)])
            pltpu.sync_copy(ec_hbm.at[0, pl.ds(a, n)], idx_v.at[1, pl.ds(0, n)])
            pltpu.async_copy(x_hbm.at[idx_v.at[0, 0]], rows_v.at[bufs[0][0]],
                             sem_a)

            def body(i, carry):
                for p in range(2):
                    j = 2 * i + p
                    half, sem = bufs[p]
                    nhalf, nsem = bufs[1 - p]

                    @pl.when(j + 1 < n)
                    def _():
                        pltpu.async_copy(x_hbm.at[idx_v.at[0, j + 1]],
                                         rows_v.at[nhalf], nsem)

                    pltpu.make_async_copy(x_hbm.at[idx_v.at[0, j]],
                                          rows_v.at[half], sem).wait()
                    pltpu.sync_copy(rows_v.at[half], agg_sh.at[idx_v.at[1, j]],
                                    add=True)
                return carry

            lax.fori_loop(0, n // 2, body, 0, unroll=False)

        @pl.when(wid < nbig)
        def _():
            start = blen * wid
            for h in range(0, blen, SEC):
                section(start + h, min(SEC, blen - h))

        @pl.when(wid >= nbig)
        def _():
            start = slen * wid + (blen - slen) * nbig
            for h in range(0, slen, SEC):
                section(start + h, min(SEC, slen - h))

        if tail:
            @pl.when(wid == nbig)
            def _():
                section(ct - tail, tail)

        plsc.subcore_barrier()
        pltpu.sync_copy(agg_sh.at[my_rows], out_hbm.at[cid].at[my_rows])

    return k(x, ec)


BM = 2000
NB = N // BM


def _tc_mlp(x, parts, eps, W1, b1, g1, be1, W2, b2, g2, be2, interpret=False):
    def body(x_ref, p_ref, eps_ref, W1_ref, b1_ref, g1_ref, be1_ref,
             W2_ref, b2_ref, g2_ref, be2_ref, o_ref, y_s, z_s, st_s):
        ph = pl.program_id(0)
        b = pl.program_id(1)
        rows = pl.ds(b * BM, BM)

        @pl.when(ph == 0)
        def _():
            agg = p_ref[0] + p_ref[1]
            h = (1.0 + eps_ref[0]) * x_ref[...] + agg
            y = jnp.dot(h, W1_ref[...], preferred_element_type=jnp.float32,
                        precision=lax.Precision.DEFAULT) + b1_ref[...]
            y_s[rows, :] = y

            @pl.when(b == 0)
            def _():
                st_s[...] = jnp.zeros_like(st_s)

            st_s[0:1, :] += jnp.sum(y, axis=0, keepdims=True)
            st_s[1:2, :] += jnp.sum(y * y, axis=0, keepdims=True)

        @pl.when(ph == 1)
        def _():
            mu = st_s[0:1, :] * (1.0 / N)
            var = st_s[1:2, :] * (1.0 / N) - mu * mu
            y = y_s[rows, :]
            y = g1_ref[...] * (y - mu) * lax.rsqrt(var + 1e-5) + be1_ref[...]
            y = jnp.maximum(y, 0.0)
            z = jnp.dot(y, W2_ref[...], preferred_element_type=jnp.float32,
                        precision=lax.Precision.DEFAULT) + b2_ref[...]
            z_s[rows, :] = z
            st_s[2:3, :] += jnp.sum(z, axis=0, keepdims=True)
            st_s[3:4, :] += jnp.sum(z * z, axis=0, keepdims=True)

        @pl.when(ph == 2)
        def _():
            mu = st_s[2:3, :] * (1.0 / N)
            var = st_s[3:4, :] * (1.0 / N) - mu * mu
            z = z_s[rows, :]
            z = g2_ref[...] * (z - mu) * lax.rsqrt(var + 1e-5) + be2_ref[...]
            o_ref[...] = jnp.maximum(z, 0.0)

    last = NB - 1
    return pl.pallas_call(
        body,
        grid=(3, NB),
        in_specs=[
            pl.BlockSpec((BM, DI), lambda ph, b: (jnp.where(ph == 0, b, last), 0)),
            pl.BlockSpec((NC, BM, DI),
                         lambda ph, b: (0, jnp.where(ph == 0, b, last), 0)),
            pl.BlockSpec(memory_space=pltpu.SMEM),
            pl.BlockSpec((DI, DO), lambda ph, b: (0, 0)),
            pl.BlockSpec((DO,), lambda ph, b: (0,)),
            pl.BlockSpec((DO,), lambda ph, b: (0,)),
            pl.BlockSpec((DO,), lambda ph, b: (0,)),
            pl.BlockSpec((DO, DO), lambda ph, b: (0, 0)),
            pl.BlockSpec((DO,), lambda ph, b: (0,)),
            pl.BlockSpec((DO,), lambda ph, b: (0,)),
            pl.BlockSpec((DO,), lambda ph, b: (0,)),
        ],
        out_specs=pl.BlockSpec((BM, DO),
                               lambda ph, b: (jnp.where(ph == 2, b, 0), 0)),
        out_shape=jax.ShapeDtypeStruct((N, DO), jnp.float32),
        scratch_shapes=[
            pltpu.VMEM((N, DO), jnp.float32),
            pltpu.VMEM((N, DO), jnp.float32),
            pltpu.VMEM((8, DO), jnp.float32),
        ],
        interpret=interpret,
    )(x, parts, eps, W1, b1, g1, be1, W2, b2, g2, be2)


def kernel(x, edge_index, eps, W1, b1, gamma1, beta1, W2, b2, gamma2, beta2):
    e = edge_index.shape[1]
    assert e % CHUNK == 0
    ec = edge_index.astype(jnp.int32).reshape(2, e // CHUNK, CHUNK)
    parts = _sc_aggregate(x, ec)
    return _tc_mlp(x, parts, eps, W1, b1, gamma1, beta1, W2, b2, gamma2, beta2)
